# baseline (speedup 1.0000x reference)
import jax
import jax.numpy as jnp
from jax import lax
from jax.experimental import pallas as pl
from jax.experimental.pallas import tpu as pltpu
from jax.experimental.pallas import tpu_sc as plsc

VOCAB = 30522
MAX_POS = 2048
HIDDEN = 768
BATCH = 4
SEQ = 2048

NC = 2
NS = 16
NW = NC * NS
LANES = 16
NVEC = HIDDEN // LANES
WPOS = SEQ // NW
CHUNK_P = 16
NCHUNK = WPOS // CHUNK_P
ROWS_C = CHUNK_P * BATCH


def _hsum16(x):
    dnums = lax.GatherDimensionNumbers(
        offset_dims=(), collapsed_slice_dims=(0,), start_index_map=(0,))
    for sh in (8, 4, 2, 1):
        idx = lax.iota(jnp.int32, LANES) ^ sh
        x = x + lax.gather(x, idx[:, None], dnums, (1,),
                           mode=lax.GatherScatterMode.PROMISE_IN_BOUNDS)
    return x


def _rsqrt16(v):
    bits = plsc.bitcast(v, jnp.int32)
    y = plsc.bitcast(jnp.int32(0x5F3759DF) - (bits >> 1), jnp.float32)
    half = v * 0.5
    for _ in range(3):
        y = y * (1.5 - half * y * y)
    return y


def _tec_body(ids_hbm, word_hbm, pos_hbm, gamma_hbm, beta_hbm, out_hbm,
              idx_v, word_v, pos_v, gamma_v, beta_v, sem):
    cid = lax.axis_index("c")
    sid = lax.axis_index("s")
    wid = sid * NC + cid
    pbase = wid * WPOS

    pltpu.sync_copy(gamma_hbm, gamma_v)
    pltpu.sync_copy(beta_hbm, beta_v)

    for c in range(NCHUNK):
        pb = pbase + c * CHUNK_P
        for b in range(BATCH):
            pltpu.sync_copy(ids_hbm.at[pl.ds(b * SEQ + pb, CHUNK_P)],
                            idx_v.at[pl.ds(b * CHUNK_P, CHUNK_P)])
        pltpu.async_copy(word_hbm.at[idx_v], word_v, sem).wait()
        pltpu.sync_copy(pos_hbm.at[pl.ds(pb, CHUNK_P)], pos_v)

        def pos_body(i, _):
            zero = jnp.zeros((LANES,), jnp.float32)
            acc = [zero] * BATCH
            acc2 = [zero] * BATCH

            def sum_body(j, carry):
                a = list(carry[:BATCH])
                q = list(carry[BATCH:])
                sl = pl.ds(j * LANES, LANES)
                p = pos_v[i, sl]
                for b in range(BATCH):
                    x = word_v[b * CHUNK_P + i, sl] + p
                    word_v[b * CHUNK_P + i, sl] = x
                    a[b] = a[b] + x
                    q[b] = q[b] + x * x
                return tuple(a) + tuple(q)

            carry = lax.fori_loop(0, NVEC, sum_body,
                                  tuple(acc) + tuple(acc2), unroll=4)
            mean = [None] * BATCH
            rstd = [None] * BATCH
            for b in range(BATCH):
                mean[b] = _hsum16(carry[b]) * (1.0 / HIDDEN)
                var = _hsum16(carry[BATCH + b]) * (1.0 / HIDDEN) - mean[b] * mean[b]
                rstd[b] = _rsqrt16(var + 1e-12)

            def norm_body(j, _):
                sl = pl.ds(j * LANES, LANES)
                g = gamma_v[sl]
                bt = beta_v[sl]
                for b in range(BATCH):
                    x = word_v[b * CHUNK_P + i, sl]
                    word_v[b * CHUNK_P + i, sl] = (x - mean[b]) * rstd[b] * g + bt
                return _

            lax.fori_loop(0, NVEC, norm_body, None, unroll=4)
            return _

        lax.fori_loop(0, CHUNK_P, pos_body, None)

        for b in range(BATCH):
            pltpu.sync_copy(word_v.at[pl.ds(b * CHUNK_P, CHUNK_P)],
                            out_hbm.at[pl.ds(b * SEQ + pb, CHUNK_P)])


def kernel(input_ids, word_table, pos_table, gamma, beta):
    ids_flat = input_ids.reshape(-1).astype(jnp.int32)
    mesh = plsc.VectorSubcoreMesh(core_axis_name="c", subcore_axis_name="s")
    call = pl.kernel(
        _tec_body,
        mesh=mesh,
        out_type=jax.ShapeDtypeStruct((BATCH * SEQ, HIDDEN), jnp.float32),
        scratch_types=[
            pltpu.VMEM((ROWS_C,), jnp.int32),
            pltpu.VMEM((ROWS_C, HIDDEN), jnp.float32),
            pltpu.VMEM((CHUNK_P, HIDDEN), jnp.float32),
            pltpu.VMEM((HIDDEN,), jnp.float32),
            pltpu.VMEM((HIDDEN,), jnp.float32),
            pltpu.SemaphoreType.DMA,
        ],
        compiler_params=pltpu.CompilerParams(needs_layout_passes=False),
    )
    out = call(ids_flat, word_table, pos_table, gamma, beta)
    return out.reshape(BATCH, SEQ, HIDDEN)


# speedup vs baseline: 2.5417x; 2.5417x over previous
import jax
import jax.numpy as jnp
from jax import lax
from jax.experimental import pallas as pl
from jax.experimental.pallas import tpu as pltpu
from jax.experimental.pallas import tpu_sc as plsc

VOCAB = 30522
MAX_POS = 2048
HIDDEN = 768
BATCH = 4
SEQ = 2048

NC = 2
NS = 16
NW = NC * NS
LANES = 16
NVEC = HIDDEN // LANES
WPOS = SEQ // NW
CHUNK_P = 16
NCHUNK = WPOS // CHUNK_P
ROWS_C = CHUNK_P * BATCH


def _hsum16(x):
    dnums = lax.GatherDimensionNumbers(
        offset_dims=(), collapsed_slice_dims=(0,), start_index_map=(0,))
    for sh in (8, 4, 2, 1):
        idx = lax.iota(jnp.int32, LANES) ^ sh
        x = x + lax.gather(x, idx[:, None], dnums, (1,),
                           mode=lax.GatherScatterMode.PROMISE_IN_BOUNDS)
    return x


def _rsqrt16(v):
    bits = plsc.bitcast(v, jnp.int32)
    y = plsc.bitcast(jnp.int32(0x5F3759DF) - (bits >> 1), jnp.float32)
    half = v * 0.5
    for _ in range(3):
        y = y * (1.5 - half * y * y)
    return y


def _tec_body(ids_hbm, word_hbm, pos_hbm, gamma_hbm, beta_hbm, out_hbm,
              idx_v, word_v, pos_v, gamma_v, beta_v, sem):
    cid = lax.axis_index("c")
    sid = lax.axis_index("s")
    wid = sid * NC + cid
    pbase = wid * WPOS

    pltpu.sync_copy(gamma_hbm, gamma_v)
    pltpu.sync_copy(beta_hbm, beta_v)

    for c in range(NCHUNK):
        pb = pbase + c * CHUNK_P
        for b in range(BATCH):
            pltpu.sync_copy(ids_hbm.at[pl.ds(b * SEQ + pb, CHUNK_P)],
                            idx_v.at[pl.ds(b * CHUNK_P, CHUNK_P)])
        pltpu.async_copy(word_hbm.at[idx_v], word_v, sem).wait()
        pltpu.sync_copy(pos_hbm.at[pl.ds(pb, CHUNK_P)], pos_v)

        def pos_body(i, _):
            zero = jnp.zeros((LANES,), jnp.float32)
            init = tuple([zero] * (2 * BATCH))

            def sum_body(j, carry):
                a = list(carry[:BATCH])
                q = list(carry[BATCH:])
                sl = pl.ds(j * LANES, LANES)
                p = pos_v[i, sl]
                for b in range(BATCH):
                    x = word_v[b * CHUNK_P + i, sl] + p
                    word_v[b * CHUNK_P + i, sl] = x
                    a[b] = a[b] + x
                    q[b] = q[b] + x * x
                return tuple(a) + tuple(q)

            carry = plsc.parallel_loop(0, NVEC, unroll=4, carry=init)(sum_body)
            mean = [None] * BATCH
            rstd = [None] * BATCH
            for b in range(BATCH):
                mean[b] = _hsum16(carry[b]) * (1.0 / HIDDEN)
                var = _hsum16(carry[BATCH + b]) * (1.0 / HIDDEN) - mean[b] * mean[b]
                rstd[b] = _rsqrt16(var + 1e-12)

            def norm_body(j):
                sl = pl.ds(j * LANES, LANES)
                g = gamma_v[sl]
                bt = beta_v[sl]
                for b in range(BATCH):
                    x = word_v[b * CHUNK_P + i, sl]
                    word_v[b * CHUNK_P + i, sl] = (x - mean[b]) * rstd[b] * g + bt

            plsc.parallel_loop(0, NVEC, unroll=4)(norm_body)
            return _

        lax.fori_loop(0, CHUNK_P, pos_body, None)

        for b in range(BATCH):
            pltpu.sync_copy(word_v.at[pl.ds(b * CHUNK_P, CHUNK_P)],
                            out_hbm.at[pl.ds(b * SEQ + pb, CHUNK_P)])


def kernel(input_ids, word_table, pos_table, gamma, beta):
    ids_flat = input_ids.reshape(-1).astype(jnp.int32)
    mesh = plsc.VectorSubcoreMesh(core_axis_name="c", subcore_axis_name="s")
    call = pl.kernel(
        _tec_body,
        mesh=mesh,
        out_type=jax.ShapeDtypeStruct((BATCH * SEQ, HIDDEN), jnp.float32),
        scratch_types=[
            pltpu.VMEM((ROWS_C,), jnp.int32),
            pltpu.VMEM((ROWS_C, HIDDEN), jnp.float32),
            pltpu.VMEM((CHUNK_P, HIDDEN), jnp.float32),
            pltpu.VMEM((HIDDEN,), jnp.float32),
            pltpu.VMEM((HIDDEN,), jnp.float32),
            pltpu.SemaphoreType.DMA,
        ],
        compiler_params=pltpu.CompilerParams(needs_layout_passes=False),
    )
    out = call(ids_flat, word_table, pos_table, gamma, beta)
    return out.reshape(BATCH, SEQ, HIDDEN)


# speedup vs baseline: 3.0954x; 1.2178x over previous
import jax
import jax.numpy as jnp
from jax import lax
from jax.experimental import pallas as pl
from jax.experimental.pallas import tpu as pltpu
from jax.experimental.pallas import tpu_sc as plsc

VOCAB = 30522
MAX_POS = 2048
HIDDEN = 768
BATCH = 4
SEQ = 2048

NC = 2
NS = 16
NW = NC * NS
LANES = 16
NVEC = HIDDEN // LANES
WPOS = SEQ // NW
CHUNK_P = 16
NCHUNK = WPOS // CHUNK_P
ROWS_C = CHUNK_P * BATCH


def _hsum16(x):
    dnums = lax.GatherDimensionNumbers(
        offset_dims=(), collapsed_slice_dims=(0,), start_index_map=(0,))
    for sh in (8, 4, 2, 1):
        idx = lax.iota(jnp.int32, LANES) ^ sh
        x = x + lax.gather(x, idx[:, None], dnums, (1,),
                           mode=lax.GatherScatterMode.PROMISE_IN_BOUNDS)
    return x


def _rsqrt16(v):
    bits = plsc.bitcast(v, jnp.int32)
    y = plsc.bitcast(jnp.int32(0x5F3759DF) - (bits >> 1), jnp.float32)
    half = v * 0.5
    for _ in range(3):
        y = y * (1.5 - half * y * y)
    return y


def _tec_body(ids_hbm, word_hbm, pos_hbm, gamma_hbm, beta_hbm, out_hbm,
              idx_v0, idx_v1, word_v0, word_v1, pos_v0, pos_v1,
              gamma_v, beta_v,
              gsem0, gsem1, psem0, psem1, osem0, osem1):
    idx_v = [idx_v0, idx_v1]
    word_v = [word_v0, word_v1]
    pos_v = [pos_v0, pos_v1]
    gsem = [gsem0, gsem1]
    psem = [psem0, psem1]
    osem = [osem0, osem1]
    cid = lax.axis_index("c")
    sid = lax.axis_index("s")
    wid = sid * NC + cid
    pbase = wid * WPOS

    pltpu.sync_copy(gamma_hbm, gamma_v)
    pltpu.sync_copy(beta_hbm, beta_v)

    ghandle = [None, None]
    phandle = [None, None]
    ohandle = [None, None]

    def stage(c, buf):
        pb = pbase + c * CHUNK_P
        for b in range(BATCH):
            pltpu.sync_copy(ids_hbm.at[pl.ds(b * SEQ + pb, CHUNK_P)],
                            idx_v[buf].at[pl.ds(b * CHUNK_P, CHUNK_P)])
        ghandle[buf] = pltpu.async_copy(
            word_hbm.at[idx_v[buf]], word_v[buf], gsem[buf])
        phandle[buf] = pltpu.async_copy(
            pos_hbm.at[pl.ds(pb, CHUNK_P)], pos_v[buf], psem[buf])

    stage(0, 0)
    for c in range(NCHUNK):
        buf = c % 2
        nbuf = 1 - buf
        if c + 1 < NCHUNK:
            if ohandle[nbuf] is not None:
                for h in ohandle[nbuf]:
                    h.wait()
                ohandle[nbuf] = None
            stage(c + 1, nbuf)
        ghandle[buf].wait()
        phandle[buf].wait()
        pb = pbase + c * CHUNK_P
        word_c = word_v[buf]
        pos_c = pos_v[buf]

        def pos_body(i, _, word_v=word_c, pos_v=pos_c):
            zero = jnp.zeros((LANES,), jnp.float32)
            init = tuple([zero] * (2 * BATCH))

            def sum_body(j, carry):
                a = list(carry[:BATCH])
                q = list(carry[BATCH:])
                sl = pl.ds(j * LANES, LANES)
                p = pos_v[i, sl]
                for b in range(BATCH):
                    x = word_v[b * CHUNK_P + i, sl] + p
                    word_v[b * CHUNK_P + i, sl] = x
                    a[b] = a[b] + x
                    q[b] = q[b] + x * x
                return tuple(a) + tuple(q)

            carry = plsc.parallel_loop(0, NVEC, unroll=4, carry=init)(sum_body)
            mean = [None] * BATCH
            rstd = [None] * BATCH
            for b in range(BATCH):
                mean[b] = _hsum16(carry[b]) * (1.0 / HIDDEN)
                var = _hsum16(carry[BATCH + b]) * (1.0 / HIDDEN) - mean[b] * mean[b]
                rstd[b] = _rsqrt16(var + 1e-12)

            def norm_body(j):
                sl = pl.ds(j * LANES, LANES)
                g = gamma_v[sl]
                bt = beta_v[sl]
                for b in range(BATCH):
                    x = word_v[b * CHUNK_P + i, sl]
                    word_v[b * CHUNK_P + i, sl] = (x - mean[b]) * rstd[b] * g + bt

            plsc.parallel_loop(0, NVEC, unroll=4)(norm_body)
            return _

        lax.fori_loop(0, CHUNK_P, pos_body, None)

        ohandle[buf] = [
            pltpu.async_copy(word_c.at[pl.ds(b * CHUNK_P, CHUNK_P)],
                             out_hbm.at[pl.ds(b * SEQ + pb, CHUNK_P)],
                             osem[buf])
            for b in range(BATCH)
        ]

    for hs in ohandle:
        if hs is not None:
            for h in hs:
                h.wait()


def kernel(input_ids, word_table, pos_table, gamma, beta):
    ids_flat = input_ids.reshape(-1).astype(jnp.int32)
    mesh = plsc.VectorSubcoreMesh(core_axis_name="c", subcore_axis_name="s")
    call = pl.kernel(
        _tec_body,
        mesh=mesh,
        out_type=jax.ShapeDtypeStruct((BATCH * SEQ, HIDDEN), jnp.float32),
        scratch_types=[
            pltpu.VMEM((ROWS_C,), jnp.int32),
            pltpu.VMEM((ROWS_C,), jnp.int32),
            pltpu.VMEM((ROWS_C, HIDDEN), jnp.float32),
            pltpu.VMEM((ROWS_C, HIDDEN), jnp.float32),
            pltpu.VMEM((CHUNK_P, HIDDEN), jnp.float32),
            pltpu.VMEM((CHUNK_P, HIDDEN), jnp.float32),
            pltpu.VMEM((HIDDEN,), jnp.float32),
            pltpu.VMEM((HIDDEN,), jnp.float32),
            pltpu.SemaphoreType.DMA,
            pltpu.SemaphoreType.DMA,
            pltpu.SemaphoreType.DMA,
            pltpu.SemaphoreType.DMA,
            pltpu.SemaphoreType.DMA,
            pltpu.SemaphoreType.DMA,
        ],
        compiler_params=pltpu.CompilerParams(needs_layout_passes=False),
    )
    out = call(ids_flat, word_table, pos_table, gamma, beta)
    return out.reshape(BATCH, SEQ, HIDDEN)
